# baseline (speedup 1.0000x reference)
import functools

import jax
import jax.numpy as jnp
from jax.experimental import pallas as pl

N = 16384
K = 30
HID = 100
DT = 0.5
CHUNK = 2048
MLP_B = 2048


def _mlp_kernel(u0_ref, s0_ref, a0_ref, b0_ref, g0_ref,
                w1u_ref, w1s_ref, b1_ref, W2T_ref, b2_ref,
                u1_ref, s1_ref, al_ref, be_ref, ga_ref):
    u0 = u0_ref[...]
    s0 = s0_ref[...]
    w1u = w1u_ref[...]
    w1s = w1s_ref[...]
    b1 = b1_ref[...]
    h = w1u * u0 + w1s * s0 + b1
    h = jnp.maximum(h, 0.0)
    z = jnp.dot(W2T_ref[...], h, preferred_element_type=jnp.float32) + b2_ref[...]
    rates = jnp.maximum(z, 0.0) + jnp.log1p(jnp.exp(-jnp.abs(z)))
    alphas = rates[0:1, :] * a0_ref[...]
    beta = rates[1:2, :] * b0_ref[...]
    gamma = rates[2:3, :] * g0_ref[...]
    u1 = u0 + (alphas - beta * u0) * DT
    s1 = s0 + (beta * u0 - gamma * s0) * DT
    u1_ref[...] = u1
    s1_ref[...] = s1
    al_ref[...] = alphas
    be_ref[...] = beta
    ga_ref[...] = gamma


def _mlp_euler(u0, s0, alpha0, beta0, gamma0, W1, b1, W2, b2):
    row = lambda x: x.reshape(1, N)
    vec_spec = pl.BlockSpec((1, MLP_B), lambda i: (0, i))
    full = lambda r, c: pl.BlockSpec((r, c), lambda i: (0, 0))
    outs = pl.pallas_call(
        _mlp_kernel,
        grid=(N // MLP_B,),
        in_specs=[vec_spec] * 5 + [full(HID, 1), full(HID, 1), full(HID, 1),
                                   full(3, HID), full(3, 1)],
        out_specs=[vec_spec] * 5,
        out_shape=[jax.ShapeDtypeStruct((1, N), jnp.float32)] * 5,
    )(row(u0), row(s0), row(alpha0), row(beta0), row(gamma0),
      W1[0].reshape(HID, 1), W1[1].reshape(HID, 1), b1.reshape(HID, 1),
      W2.T, b2.reshape(3, 1))
    return [o.reshape(N) for o in outs]


def _knn_indices(points, k):
    sq = jnp.sum(points ** 2, axis=1)
    idx_chunks = []
    n = points.shape[0]
    for i in range(0, n, CHUNK):
        q = jax.lax.dynamic_slice(points, (i, 0), (CHUNK, 2))
        qsq = jnp.sum(q ** 2, axis=1)
        d2 = qsq[:, None] + sq[None, :] - 2.0 * (q @ points.T)
        _, idx = jax.lax.top_k(-d2, k)
        idx_chunks.append(idx)
    return jnp.concatenate(idx_chunks, axis=0)


def kernel(u0, s0, alpha0, beta0, gamma0, embedding1, embedding2, W1, b1, W2, b2):
    points = jnp.stack([embedding1, embedding2], axis=1)
    indices = _knn_indices(points, K)
    u1, s1, alphas, beta, gamma = _mlp_euler(u0, s0, alpha0, beta0, gamma0, W1, b1, W2, b2)
    uv = u1 - u0
    sv = s1 - s0
    neigh = indices.T[1:]
    unv = u0[neigh] - u0[None, :]
    snv = s0[neigh] - s0[None, :]
    den = jnp.sqrt(unv ** 2 + snv ** 2) * jnp.sqrt(uv ** 2 + sv ** 2)[None, :]
    den_safe = jnp.where(den == 0.0, 1.0, den)
    cosine = jnp.where(den != 0.0, (unv * uv[None, :] + snv * sv[None, :]) / den_safe, 1.0)
    cosine_max = jnp.max(cosine, axis=0)
    cost1 = 1.0 - cosine_max
    cost_fin = jnp.mean(cost1)
    return (cost_fin, u1, s1, alphas, beta, gamma)


# speedup vs baseline: 6.6997x; 6.6997x over previous
import jax
import jax.numpy as jnp
import numpy as np
from jax.experimental import pallas as pl
from jax.experimental.pallas import tpu as pltpu

N = 16384
K = 30
HID = 100
DT = 0.5
Q = 256
GRID = N // Q

_SIGN = np.int32(np.uint32(0x80000000).view(np.int32))
_MAG = np.int32(0x7FFFFFFF)
_HI16 = np.int32(np.uint32(0xFFFF0000).view(np.int32))


def _bf16r(x):
    i = jax.lax.bitcast_convert_type(x, jnp.int32)
    r = i + np.int32(0x7FFF) + ((i >> 16) & np.int32(1))
    return jax.lax.bitcast_convert_type(r & _HI16, jnp.float32)


def _fused_kernel(pxr, pyr, sqr, u0r, s0r,
                  pxc, pyc, sqc, u0c, s0c, a0c, b0c, g0c,
                  w1u, w1s, b1r, W2, b2r,
                  u1_o, s1_o, al_o, be_o, ga_o, cost_o,
                  key_s):
    u0q = u0c[...]
    s0q = s0c[...]

    h = jnp.maximum(u0q * w1u[...] + s0q * w1s[...] + b1r[...], 0.0)
    z = jnp.dot(h, W2[...], preferred_element_type=jnp.float32) + b2r[...]
    rates = jnp.maximum(z, 0.0) + jnp.log1p(jnp.exp(-jnp.abs(z)))
    alphas = rates[:, 0:1] * a0c[...]
    beta = rates[:, 1:2] * b0c[...]
    gamma = rates[:, 2:3] * g0c[...]
    uv = (alphas - beta * u0q) * DT
    sv = (beta * u0q - gamma * s0q) * DT
    u1_o[...] = u0q + uv
    s1_o[...] = s0q + sv
    al_o[...] = alphas
    be_o[...] = beta
    ga_o[...] = gamma

    qp = (_bf16r(pxc[...]) * _bf16r(pxr[...]) +
          _bf16r(pyc[...]) * _bf16r(pyr[...]))
    d2 = (sqc[...] + sqr[...]) - 2.0 * qp
    kk = jax.lax.bitcast_convert_type(d2, jnp.int32)
    skey = jnp.where(kk < 0, kk ^ _MAG, kk)
    key_s[...] = skey

    p = jnp.zeros((Q, 1), jnp.int32)
    kf = np.float32(K)
    for b in range(31, -1, -1):
        bit = _SIGN if b == 31 else np.int32(1 << b)
        cand_u = p | _MAG if b == 31 else p | np.int32((1 << b) - 1)
        scand = cand_u ^ _SIGN
        cnt = jnp.sum((key_s[...] <= scand).astype(jnp.float32), axis=1,
                      keepdims=True)
        p = jnp.where(cnt >= kf, p, p | bit)
    sT = p ^ _SIGN

    sk = key_s[...]
    m1 = jnp.min(sk, axis=1, keepdims=True)
    mask = (sk <= sT) & (sk > m1)
    unv = u0r[...] - u0q
    snv = s0r[...] - s0q
    den = jnp.sqrt(unv * unv + snv * snv) * jnp.sqrt(uv * uv + sv * sv)
    num = unv * uv + snv * sv
    cos = jnp.where(den != 0.0, num / jnp.where(den == 0.0, 1.0, den), 1.0)
    cos_max = jnp.max(jnp.where(mask, cos, -2.0), axis=1, keepdims=True)
    cost_o[...] = jnp.sum(1.0 - cos_max).reshape(1, 1, 1)


def kernel(u0, s0, alpha0, beta0, gamma0, embedding1, embedding2, W1, b1, W2, b2):
    points = jnp.stack([embedding1, embedding2], axis=1)
    sq = jnp.sum(points ** 2, axis=1)
    e1b = embedding1
    e2b = embedding2

    row = lambda x: x.reshape(1, N)
    col = lambda x: x.reshape(N, 1)
    full = lambda *shape: pl.BlockSpec(shape, lambda i: (0,) * len(shape))
    colspec = pl.BlockSpec((Q, 1), lambda i: (i, 0))

    outs = pl.pallas_call(
        _fused_kernel,
        grid=(GRID,),
        in_specs=[full(1, N)] * 5 + [colspec] * 8 +
                 [full(1, HID), full(1, HID), full(1, HID),
                  full(HID, 3), full(1, 3)],
        out_specs=[colspec] * 5 + [pl.BlockSpec((1, 1, 1), lambda i: (i, 0, 0))],
        out_shape=[jax.ShapeDtypeStruct((N, 1), jnp.float32)] * 5 +
                  [jax.ShapeDtypeStruct((GRID, 1, 1), jnp.float32)],
        scratch_shapes=[pltpu.VMEM((Q, N), jnp.int32)],
    )(row(e1b), row(e2b), row(sq), row(u0), row(s0),
      col(e1b), col(e2b), col(sq), col(u0), col(s0),
      col(alpha0), col(beta0), col(gamma0),
      W1[0].reshape(1, HID), W1[1].reshape(1, HID), b1.reshape(1, HID),
      W2, b2.reshape(1, 3))

    u1, s1, alphas, beta, gamma, parts = outs
    cost_fin = jnp.sum(parts) / np.float32(N)
    return (cost_fin, u1.reshape(N), s1.reshape(N), alphas.reshape(N),
            beta.reshape(N), gamma.reshape(N))
